# baseline (speedup 1.0000x reference)
import functools

import jax
import jax.numpy as jnp
from jax import lax
from jax.experimental import pallas as pl
from jax.experimental.pallas import tpu as pltpu
from jax.experimental.pallas import tpu_sc as plsc

_BB = 256
_NC = 8


def _score_kernel(a_ref, n_ref, idx_ref):
    a = a_ref[...]
    BB, N, D = n_ref.shape
    ab = a[:, None, :]
    cols = []
    for k in range(N // _NC):
        nk = n_ref[:, k * _NC:(k + 1) * _NC, :]
        cols.append(1.0 - jnp.sum(nk * ab, axis=2))
    dist = jnp.concatenate(cols, axis=1)
    dmin = jnp.min(dist, axis=1, keepdims=True)
    nidx = jax.lax.broadcasted_iota(jnp.int32, (BB, N), 1)
    idx = jnp.min(jnp.where(dist <= dmin, nidx, N), axis=1, keepdims=True)
    row = (jax.lax.broadcasted_iota(jnp.int32, (BB, 1), 0)
           + pl.program_id(0) * BB + _SB)
    idx_ref[...] = row * N + idx


def _tc_scores(anchor, negatives, start):
    B, N, D = negatives.shape
    off = start // _BB
    return pl.pallas_call(
        _score_kernel,
        grid=((B - start) // _BB,),
        in_specs=[
            pl.BlockSpec((_BB, D), lambda i: (i + off, 0)),
            pl.BlockSpec((_BB, N, D), lambda i: (i + off, 0, 0)),
        ],
        out_specs=pl.BlockSpec((_BB, 1), lambda i: (i, 0)),
        out_shape=jax.ShapeDtypeStruct((B - start, 1), jnp.int32),
    )(anchor, negatives)


def _sc_gather(table, fidx):
    BN, D = table.shape
    B = fidx.shape[0]
    info = plsc.get_sparse_core_info()
    nw = info.num_cores * info.num_subcores
    b_per_w = B // nw
    mesh = plsc.VectorSubcoreMesh(core_axis_name="c", subcore_axis_name="s")

    @functools.partial(
        pl.kernel,
        mesh=mesh,
        out_type=jax.ShapeDtypeStruct((B, D), jnp.float32),
        scratch_types=[
            pltpu.VMEM((b_per_w,), jnp.int32),
            pltpu.VMEM((b_per_w, D), jnp.float32),
            pltpu.SemaphoreType.DMA,
        ],
    )
    def k(table_hbm, idx_hbm, out_hbm, idx_v, rows_v, sem):
        wid = lax.axis_index("s") * info.num_cores + lax.axis_index("c")
        base = wid * b_per_w
        pltpu.sync_copy(idx_hbm.at[pl.ds(base, b_per_w)], idx_v)
        pltpu.async_copy(table_hbm.at[idx_v], rows_v, sem).wait()
        pltpu.sync_copy(rows_v, out_hbm.at[pl.ds(base, b_per_w)])

    return k(table, fidx)


_SB = 512


def _sc_mine(anchor, negatives, SB):
    _, N, D = negatives.shape
    info = plsc.get_sparse_core_info()
    nw = info.num_cores * info.num_subcores
    R = SB // nw
    NCH = D // 16
    mesh = plsc.VectorSubcoreMesh(core_axis_name="c", subcore_axis_name="s")

    NG = (N + 15) // 16
    NP = NG * 16

    @functools.partial(
        pl.kernel,
        mesh=mesh,
        out_type=jax.ShapeDtypeStruct((SB, D), jnp.float32),
        scratch_types=[
            pltpu.VMEM((R, D), jnp.float32),
            pltpu.VMEM((N, D), jnp.float32),
            pltpu.VMEM((N, D), jnp.float32),
            pltpu.SemaphoreType.DMA,
            pltpu.SemaphoreType.DMA,
            pltpu.SemaphoreType.DMA,
        ],
    )
    def k(neg_hbm, a_hbm, out_hbm, a_v, buf0, buf1, sem0, sem1, sem_a):
        wid = lax.axis_index("s") * info.num_cores + lax.axis_index("c")
        base = wid * R
        pltpu.async_copy(a_hbm.at[pl.ds(base, R)], a_v, sem_a).wait()
        pltpu.async_copy(neg_hbm.at[base], buf0, sem0)
        lane = lax.iota(jnp.int32, 16)
        perms = [jnp.bitwise_xor(lane, sh) for sh in (8, 4, 2, 1)]
        inf_v = jnp.full((16,), jnp.inf, dtype=jnp.float32)
        dnums = lax.GatherDimensionNumbers(
            offset_dims=(), collapsed_slice_dims=(0,), start_index_map=(0,))

        def _allsum(acc):
            for p in perms:
                g = lax.gather(acc, p[:, None], dnums, (1,),
                               mode=lax.GatherScatterMode.PROMISE_IN_BOUNDS)
                acc = acc + g
            return acc

        GS = 8

        def mine_row(r, buf):
            a_c = [a_v[r, pl.ds(c * 16, 16)] for c in range(NCH)]

            def g_body(g, carry):
                bestd, besti = carry
                merged = inf_v
                for i in range(GS):
                    n = g * GS + i
                    acc = buf[n, pl.ds(0, 16)] * a_c[0]
                    for c in range(1, NCH):
                        acc = acc + buf[n, pl.ds(c * 16, 16)] * a_c[c]
                    d = 1.0 - _allsum(acc)
                    merged = jnp.where(lane == i, d, merged)
                take = merged < bestd
                nv = lane + g * GS
                return (jnp.where(take, merged, bestd),
                        jnp.where(take, nv, besti))

            bestd, besti = lax.fori_loop(
                0, N // GS, g_body, (inf_v, jnp.zeros((16,), jnp.int32)))
            bd, bi = bestd[0], besti[0]
            for l in range(1, 16):
                dl, il = bestd[l], besti[l]
                take = (dl < bd) | ((dl == bd) & (il < bi))
                bd = lax.select(take, dl, bd)
                bi = lax.select(take, il, bi)
            pltpu.sync_copy(buf.at[bi], out_hbm.at[base + r])

        last = base + R - 1

        def pair_body(p, _):
            r0 = 2 * p
            pltpu.async_copy(
                neg_hbm.at[jnp.minimum(base + r0 + 1, last)], buf1, sem1)
            pltpu.make_async_copy(neg_hbm.at[base], buf0, sem0).wait()
            mine_row(r0, buf0)
            pltpu.async_copy(
                neg_hbm.at[jnp.minimum(base + r0 + 2, last)], buf0, sem0)
            pltpu.make_async_copy(neg_hbm.at[base], buf1, sem1).wait()
            mine_row(r0 + 1, buf1)
            return 0

        lax.fori_loop(0, R // 2, pair_body, 0)
        pltpu.make_async_copy(neg_hbm.at[base], buf0, sem0).wait()

    return k(negatives, anchor)


def kernel(anchor, negatives):
    B, N, D = negatives.shape
    out_sc = _sc_mine(anchor, negatives, _SB)
    fidx = _tc_scores(anchor, negatives, _SB).reshape((B - _SB,))
    table = negatives.reshape((B * N, D))
    out_tc = _sc_gather(table, fidx)
    return jnp.concatenate([out_sc, out_tc], axis=0)


# speedup vs baseline: 1.0327x; 1.0327x over previous
import functools

import jax
import jax.numpy as jnp
from jax import lax
from jax.experimental import pallas as pl
from jax.experimental.pallas import tpu as pltpu
from jax.experimental.pallas import tpu_sc as plsc

_BB = 256
_NC = 8


def _score_kernel(a_ref, n_ref, idx_ref):
    a = a_ref[...]
    BB, N, D = n_ref.shape
    ab = a[:, None, :]
    cols = []
    for k in range(N // _NC):
        nk = n_ref[:, k * _NC:(k + 1) * _NC, :]
        cols.append(1.0 - jnp.sum(nk * ab, axis=2))
    dist = jnp.concatenate(cols, axis=1)
    dmin = jnp.min(dist, axis=1, keepdims=True)
    nidx = jax.lax.broadcasted_iota(jnp.int32, (BB, N), 1)
    idx = jnp.min(jnp.where(dist <= dmin, nidx, N), axis=1, keepdims=True)
    row = (jax.lax.broadcasted_iota(jnp.int32, (BB, 1), 0)
           + pl.program_id(0) * BB)
    idx_ref[...] = row * N + idx


def _tc_scores(anchor, negatives):
    B, N, D = negatives.shape
    return pl.pallas_call(
        _score_kernel,
        grid=(B // _BB,),
        in_specs=[
            pl.BlockSpec((_BB, D), lambda i: (i, 0)),
            pl.BlockSpec((_BB, N, D), lambda i: (i, 0, 0)),
        ],
        out_specs=pl.BlockSpec((_BB, 1), lambda i: (i, 0)),
        out_shape=jax.ShapeDtypeStruct((B, 1), jnp.int32),
    )(anchor, negatives)


def _sc_gather(table, fidx):
    BN, D = table.shape
    B = fidx.shape[0]
    info = plsc.get_sparse_core_info()
    nw = info.num_cores * info.num_subcores
    b_per_w = B // nw
    mesh = plsc.VectorSubcoreMesh(core_axis_name="c", subcore_axis_name="s")

    @functools.partial(
        pl.kernel,
        mesh=mesh,
        out_type=jax.ShapeDtypeStruct((B, D), jnp.float32),
        scratch_types=[
            pltpu.VMEM((b_per_w,), jnp.int32),
            pltpu.VMEM((b_per_w, D), jnp.float32),
            pltpu.SemaphoreType.DMA,
        ],
    )
    def k(table_hbm, idx_hbm, out_hbm, idx_v, rows_v, sem):
        wid = lax.axis_index("s") * info.num_cores + lax.axis_index("c")
        base = wid * b_per_w
        pltpu.sync_copy(idx_hbm.at[pl.ds(base, b_per_w)], idx_v)
        pltpu.async_copy(table_hbm.at[idx_v], rows_v, sem).wait()
        pltpu.sync_copy(rows_v, out_hbm.at[pl.ds(base, b_per_w)])

    return k(table, fidx)


def kernel(anchor, negatives):
    B, N, D = negatives.shape
    fidx = _tc_scores(anchor, negatives).reshape((B,))
    table = negatives.reshape((B * N, D))
    return _sc_gather(table, fidx)
